# baseline (speedup 1.0000x reference)
import jax
import jax.numpy as jnp
from jax.experimental import pallas as pl

B = 1024
BLK = 256
INPUT_DIM = 768
H1, H2 = 512, 256
LATENT = 64
NQ = 4
K = 512
NCAND = 4

f32 = jnp.float32
bf16 = jnp.bfloat16
i32 = jnp.int32


def _tree8_rows(diff):
    ax = diff.ndim - 1
    sq = diff * diff
    t = sq + jnp.roll(sq, -4, axis=ax)
    t = t + jnp.roll(t, -2, axis=ax)
    t = t + jnp.roll(t, -1, axis=ax)
    acc = t[..., 0:1]
    for s in range(1, 8):
        acc = acc + t[..., 8 * s:8 * s + 1]
    return acc


def _body(x, w1, b1, w2, b2, muW, mub, lvW, lvb, cbTh, cbTl, cbs, en_ref,
          d1W, d1b, d2W, d2b, d3W, d3b, eps,
          xr_out, mu_out, lv_out, qs_out, codes_out):
    h = jnp.maximum(jnp.dot(x[:], w1[:], preferred_element_type=f32) + b1[:], 0.0)
    h = jnp.maximum(jnp.dot(h, w2[:], preferred_element_type=f32) + b2[:], 0.0)
    mu = jnp.dot(h, muW[:], preferred_element_type=f32) + mub[:]
    lv = jnp.dot(h, lvW[:], preferred_element_type=f32) + lvb[:]
    mu_out[:] = mu
    lv_out[:] = lv
    z = mu + eps[:] * jnp.exp(0.5 * lv)

    residual = z
    qsum = jnp.zeros_like(z)
    codes = []
    kiota = jax.lax.broadcasted_iota(i32, (BLK, K), 1)
    for i in range(NQ):
        eTh = cbTh[i]
        eTl = cbTl[i]
        esplit = cbs[i]
        en = en_ref[i]
        r_hi = residual.astype(bf16)
        r_lo = (residual - r_hi.astype(f32)).astype(bf16)
        s = en - 2.0 * (jnp.dot(r_hi, eTh, preferred_element_type=f32)
                        + (jnp.dot(r_hi, eTl, preferred_element_type=f32)
                           + jnp.dot(r_lo, eTh, preferred_element_type=f32)))
        sp = jax.lax.bitcast_convert_type(
            (jax.lax.bitcast_convert_type(s, i32) & jnp.int32(-512)) | kiota, f32)
        ks = []
        for t in range(NCAND):
            mn = jnp.min(sp, axis=1, keepdims=True)
            k_t = jax.lax.bitcast_convert_type(mn, i32) & jnp.int32(511)
            hit = sp == mn
            sp = jnp.where(hit, jnp.inf, sp)
            ks.append(k_t[:, 0])
        kks = jnp.concatenate(ks, axis=0)
        oh_all = (jax.lax.broadcasted_iota(i32, (NCAND * BLK, K), 1)
                  == kks[:, None]).astype(bf16)
        g = jnp.dot(oh_all, esplit, preferred_element_type=f32)
        e_all = (g[:, :LATENT] + g[:, LATENT:2 * LATENT]) + g[:, 2 * LATENT:]
        e_r = e_all.reshape(NCAND, BLK, LATENT)
        d_all = _tree8_rows(residual[None, :, :] - e_r)
        best_d = d_all[0, :, 0]
        best_k = ks[0]
        best_e = e_r[0]
        for t in range(1, NCAND):
            d_t = d_all[t, :, 0]
            k_t = ks[t]
            e_t = e_r[t]
            better = (d_t < best_d) | ((d_t == best_d) & (k_t < best_k))
            best_d = jnp.where(better, d_t, best_d)
            best_k = jnp.where(better, k_t, best_k)
            best_e = jnp.where(better[:, None], e_t, best_e)
        qsum = qsum + best_e
        residual = residual - best_e
        codes.append(best_k)
    qs_out[:] = qsum
    codes_out[:] = jnp.stack(codes, axis=1)

    d = jnp.maximum(jnp.dot(qsum, d1W[:], preferred_element_type=f32) + d1b[:], 0.0)
    d = jnp.maximum(jnp.dot(d, d2W[:], preferred_element_type=f32) + d2b[:], 0.0)
    xr_out[:] = jnp.dot(d, d3W[:], preferred_element_type=f32) + d3b[:]


def _split3(cb):
    bits = jax.lax.bitcast_convert_type(cb, i32)
    mask = jnp.int32(-65536)
    e1 = jax.lax.bitcast_convert_type(bits & mask, f32)
    r1 = cb - e1
    r1b = jax.lax.bitcast_convert_type(r1, i32)
    e2 = jax.lax.bitcast_convert_type(r1b & mask, f32)
    e3 = r1 - e2
    return jnp.concatenate([e1, e2, e3], axis=-1).astype(bf16)


def _rep(shape):
    nd = len(shape)
    return pl.BlockSpec(shape, lambda i, _n=nd: (0,) * _n)


def _rows(shape):
    rest = shape[1:]
    return pl.BlockSpec((BLK,) + rest,
                        lambda i, _n=len(rest): (i,) + (0,) * _n)


@jax.jit
def kernel(x, enc_W1, enc_b1, enc_W2, enc_b2, mu_W, mu_b, lv_W, lv_b,
           codebooks, dec_W1, dec_b1, dec_W2, dec_b2, dec_W3, dec_b3, eps):
    outs = (
        jax.ShapeDtypeStruct((B, INPUT_DIM), f32),
        jax.ShapeDtypeStruct((B, LATENT), f32),
        jax.ShapeDtypeStruct((B, LATENT), f32),
        jax.ShapeDtypeStruct((B, LATENT), f32),
        jax.ShapeDtypeStruct((B, NQ), jnp.int32),
    )
    cbT = codebooks.transpose(0, 2, 1)
    cbTh = cbT.astype(bf16)
    cbTl = (cbT - cbTh.astype(f32)).astype(bf16)
    cbs = _split3(codebooks)
    en = jnp.sum(codebooks * codebooks, axis=2)[:, None, :]
    b1 = enc_b1.reshape(1, -1)
    b2 = enc_b2.reshape(1, -1)
    mub = mu_b.reshape(1, -1)
    lvb = lv_b.reshape(1, -1)
    db1 = dec_b1.reshape(1, -1)
    db2 = dec_b2.reshape(1, -1)
    db3 = dec_b3.reshape(1, -1)
    in_specs = [
        _rows(x.shape),
        _rep(enc_W1.shape), _rep(b1.shape), _rep(enc_W2.shape), _rep(b2.shape),
        _rep(mu_W.shape), _rep(mub.shape), _rep(lv_W.shape), _rep(lvb.shape),
        _rep(cbTh.shape), _rep(cbTl.shape), _rep(cbs.shape), _rep(en.shape),
        _rep(dec_W1.shape), _rep(db1.shape), _rep(dec_W2.shape), _rep(db2.shape),
        _rep(dec_W3.shape), _rep(db3.shape),
        _rows(eps.shape),
    ]
    out_specs = tuple(_rows(o.shape) for o in outs)
    return pl.pallas_call(
        _body,
        grid=(B // BLK,),
        in_specs=in_specs,
        out_specs=out_specs,
        out_shape=outs,
    )(x, enc_W1, b1, enc_W2, b2, mu_W, mub, lv_W, lvb,
      cbTh, cbTl, cbs, en, dec_W1, db1, dec_W2, db2, dec_W3, db3, eps)


# speedup vs baseline: 1.1956x; 1.1956x over previous
import jax
import jax.numpy as jnp
from jax.experimental import pallas as pl

B = 1024
BLK = 256
INPUT_DIM = 768
H1, H2 = 512, 256
LATENT = 64
NQ = 4
K = 512
NCAND = 4

f32 = jnp.float32
bf16 = jnp.bfloat16
i32 = jnp.int32


def _tree8_rows(diff):
    ax = diff.ndim - 1
    sq = diff * diff
    t = sq + jnp.roll(sq, -4, axis=ax)
    t = t + jnp.roll(t, -2, axis=ax)
    t = t + jnp.roll(t, -1, axis=ax)
    acc = t[..., 0:1]
    for s in range(1, 8):
        acc = acc + t[..., 8 * s:8 * s + 1]
    return acc


def _body(x, w1, b1, w2, b2, muW, mub, lvW, lvb, cbTh, cbTl, cbs, en_ref,
          d1W, d1b, d2W, d2b, d3W, d3b, eps,
          xr_out, mu_out, lv_out, qs_out, codes_out):
    h = jnp.maximum(jnp.dot(x[:], w1[:], preferred_element_type=f32) + b1[:], 0.0)
    h = jnp.maximum(jnp.dot(h, w2[:], preferred_element_type=f32) + b2[:], 0.0)
    mu = jnp.dot(h, muW[:], preferred_element_type=f32) + mub[:]
    lv = jnp.dot(h, lvW[:], preferred_element_type=f32) + lvb[:]
    mu_out[:] = mu
    lv_out[:] = lv
    z = mu + eps[:] * jnp.exp(0.5 * lv)

    residual = z
    qsum = jnp.zeros_like(z)
    codes = []
    kiota = jax.lax.broadcasted_iota(i32, (BLK, K), 1)
    for i in range(NQ):
        eTh = cbTh[i]
        eTl = cbTl[i]
        esplit = cbs[i]
        en = en_ref[i]
        r_hi = residual.astype(bf16)
        r_lo = (residual - r_hi.astype(f32)).astype(bf16)
        s = en - 2.0 * (jnp.dot(r_hi, eTh, preferred_element_type=f32)
                        + (jnp.dot(r_hi, eTl, preferred_element_type=f32)
                           + jnp.dot(r_lo, eTh, preferred_element_type=f32)))
        sp = jax.lax.bitcast_convert_type(
            (jax.lax.bitcast_convert_type(s, i32) & jnp.int32(-512)) | kiota, f32)
        ks, ohs = [], []
        for t in range(NCAND):
            mn = jnp.min(sp, axis=1, keepdims=True)
            k_t = jax.lax.bitcast_convert_type(mn, i32) & jnp.int32(511)
            hit = sp == mn
            sp = jnp.where(hit, jnp.inf, sp)
            ks.append(k_t[:, 0])
            ohs.append(hit.astype(bf16))
        oh_all = jnp.concatenate(ohs, axis=0)
        g = jnp.dot(oh_all, esplit, preferred_element_type=f32)
        e_all = (g[:, :LATENT] + g[:, LATENT:2 * LATENT]) + g[:, 2 * LATENT:]
        best_d = best_k = best_e = None
        for t in range(NCAND):
            e_t = e_all[t * BLK:(t + 1) * BLK]
            d_t = _tree8_rows(residual - e_t)[:, 0]
            k_t = ks[t]
            if best_d is None:
                best_d, best_k, best_e = d_t, k_t, e_t
            else:
                better = (d_t < best_d) | ((d_t == best_d) & (k_t < best_k))
                best_d = jnp.where(better, d_t, best_d)
                best_k = jnp.where(better, k_t, best_k)
                best_e = jnp.where(better[:, None], e_t, best_e)
        qsum = qsum + best_e
        residual = residual - best_e
        codes.append(best_k)
    qs_out[:] = qsum
    codes_out[:] = jnp.stack(codes, axis=1)

    d = jnp.maximum(jnp.dot(qsum, d1W[:], preferred_element_type=f32) + d1b[:], 0.0)
    d = jnp.maximum(jnp.dot(d, d2W[:], preferred_element_type=f32) + d2b[:], 0.0)
    xr_out[:] = jnp.dot(d, d3W[:], preferred_element_type=f32) + d3b[:]


def _split3(cb):
    bits = jax.lax.bitcast_convert_type(cb, i32)
    mask = jnp.int32(-65536)
    e1 = jax.lax.bitcast_convert_type(bits & mask, f32)
    r1 = cb - e1
    r1b = jax.lax.bitcast_convert_type(r1, i32)
    e2 = jax.lax.bitcast_convert_type(r1b & mask, f32)
    e3 = r1 - e2
    return jnp.concatenate([e1, e2, e3], axis=-1).astype(bf16)


def _rep(shape):
    nd = len(shape)
    return pl.BlockSpec(shape, lambda i, _n=nd: (0,) * _n)


def _rows(shape):
    rest = shape[1:]
    return pl.BlockSpec((BLK,) + rest,
                        lambda i, _n=len(rest): (i,) + (0,) * _n)


@jax.jit
def kernel(x, enc_W1, enc_b1, enc_W2, enc_b2, mu_W, mu_b, lv_W, lv_b,
           codebooks, dec_W1, dec_b1, dec_W2, dec_b2, dec_W3, dec_b3, eps):
    outs = (
        jax.ShapeDtypeStruct((B, INPUT_DIM), f32),
        jax.ShapeDtypeStruct((B, LATENT), f32),
        jax.ShapeDtypeStruct((B, LATENT), f32),
        jax.ShapeDtypeStruct((B, LATENT), f32),
        jax.ShapeDtypeStruct((B, NQ), jnp.int32),
    )
    cbT = codebooks.transpose(0, 2, 1)
    cbTh = cbT.astype(bf16)
    cbTl = (cbT - cbTh.astype(f32)).astype(bf16)
    cbs = _split3(codebooks)
    en = jnp.sum(codebooks * codebooks, axis=2)[:, None, :]
    b1 = enc_b1.reshape(1, -1)
    b2 = enc_b2.reshape(1, -1)
    mub = mu_b.reshape(1, -1)
    lvb = lv_b.reshape(1, -1)
    db1 = dec_b1.reshape(1, -1)
    db2 = dec_b2.reshape(1, -1)
    db3 = dec_b3.reshape(1, -1)
    in_specs = [
        _rows(x.shape),
        _rep(enc_W1.shape), _rep(b1.shape), _rep(enc_W2.shape), _rep(b2.shape),
        _rep(mu_W.shape), _rep(mub.shape), _rep(lv_W.shape), _rep(lvb.shape),
        _rep(cbTh.shape), _rep(cbTl.shape), _rep(cbs.shape), _rep(en.shape),
        _rep(dec_W1.shape), _rep(db1.shape), _rep(dec_W2.shape), _rep(db2.shape),
        _rep(dec_W3.shape), _rep(db3.shape),
        _rows(eps.shape),
    ]
    out_specs = tuple(_rows(o.shape) for o in outs)
    return pl.pallas_call(
        _body,
        grid=(B // BLK,),
        in_specs=in_specs,
        out_specs=out_specs,
        out_shape=outs,
    )(x, enc_W1, b1, enc_W2, b2, mu_W, mub, lv_W, lvb,
      cbTh, cbTl, cbs, en, dec_W1, db1, dec_W2, db2, dec_W3, db3, eps)


# speedup vs baseline: 1.2947x; 1.0829x over previous
import jax
import jax.numpy as jnp
from jax.experimental import pallas as pl

B = 1024
BLK = 512
INPUT_DIM = 768
H1, H2 = 512, 256
LATENT = 64
NQ = 4
K = 512
NCAND = 4

f32 = jnp.float32
bf16 = jnp.bfloat16
i32 = jnp.int32


def _tree8_rows(diff):
    ax = diff.ndim - 1
    sq = diff * diff
    t = sq + jnp.roll(sq, -4, axis=ax)
    t = t + jnp.roll(t, -2, axis=ax)
    t = t + jnp.roll(t, -1, axis=ax)
    acc = t[..., 0:1]
    for s in range(1, 8):
        acc = acc + t[..., 8 * s:8 * s + 1]
    return acc


def _body(x, w1, b1, w2, b2, muW, mub, lvW, lvb, cbTh, cbTl, cbs, en_ref,
          d1W, d1b, d2W, d2b, d3W, d3b, eps,
          xr_out, mu_out, lv_out, qs_out, codes_out):
    h = jnp.maximum(jnp.dot(x[:], w1[:], preferred_element_type=f32) + b1[:], 0.0)
    h = jnp.maximum(jnp.dot(h, w2[:], preferred_element_type=f32) + b2[:], 0.0)
    mu = jnp.dot(h, muW[:], preferred_element_type=f32) + mub[:]
    lv = jnp.dot(h, lvW[:], preferred_element_type=f32) + lvb[:]
    mu_out[:] = mu
    lv_out[:] = lv
    z = mu + eps[:] * jnp.exp(0.5 * lv)

    residual = z
    qsum = jnp.zeros_like(z)
    codes = []
    kiota = jax.lax.broadcasted_iota(i32, (BLK, K), 1)
    for i in range(NQ):
        eTh = cbTh[i]
        eTl = cbTl[i]
        esplit = cbs[i]
        en = en_ref[i]
        r_hi = residual.astype(bf16)
        r_lo = (residual - r_hi.astype(f32)).astype(bf16)
        s = en - 2.0 * (jnp.dot(r_hi, eTh, preferred_element_type=f32)
                        + (jnp.dot(r_hi, eTl, preferred_element_type=f32)
                           + jnp.dot(r_lo, eTh, preferred_element_type=f32)))
        sp = jax.lax.bitcast_convert_type(
            (jax.lax.bitcast_convert_type(s, i32) & jnp.int32(-512)) | kiota, f32)
        ks, ohs = [], []
        for t in range(NCAND):
            mn = jnp.min(sp, axis=1, keepdims=True)
            k_t = jax.lax.bitcast_convert_type(mn, i32) & jnp.int32(511)
            hit = sp == mn
            sp = jnp.where(hit, jnp.inf, sp)
            ks.append(k_t[:, 0])
            ohs.append(hit.astype(bf16))
        oh_all = jnp.concatenate(ohs, axis=0)
        g = jnp.dot(oh_all, esplit, preferred_element_type=f32)
        e_all = (g[:, :LATENT] + g[:, LATENT:2 * LATENT]) + g[:, 2 * LATENT:]
        r4 = jnp.concatenate([residual] * NCAND, axis=0)
        d_all = _tree8_rows(r4 - e_all)
        best_d = d_all[0:BLK, 0]
        best_k = ks[0]
        best_e = e_all[0:BLK]
        for t in range(1, NCAND):
            d_t = d_all[t * BLK:(t + 1) * BLK, 0]
            k_t = ks[t]
            e_t = e_all[t * BLK:(t + 1) * BLK]
            better = (d_t < best_d) | ((d_t == best_d) & (k_t < best_k))
            best_d = jnp.where(better, d_t, best_d)
            best_k = jnp.where(better, k_t, best_k)
            best_e = jnp.where(better[:, None], e_t, best_e)
        qsum = qsum + best_e
        residual = residual - best_e
        codes.append(best_k)
    qs_out[:] = qsum
    codes_out[:] = jnp.stack(codes, axis=1)

    d = jnp.maximum(jnp.dot(qsum, d1W[:], preferred_element_type=f32) + d1b[:], 0.0)
    d = jnp.maximum(jnp.dot(d, d2W[:], preferred_element_type=f32) + d2b[:], 0.0)
    xr_out[:] = jnp.dot(d, d3W[:], preferred_element_type=f32) + d3b[:]


def _split3(cb):
    bits = jax.lax.bitcast_convert_type(cb, i32)
    mask = jnp.int32(-65536)
    e1 = jax.lax.bitcast_convert_type(bits & mask, f32)
    r1 = cb - e1
    r1b = jax.lax.bitcast_convert_type(r1, i32)
    e2 = jax.lax.bitcast_convert_type(r1b & mask, f32)
    e3 = r1 - e2
    return jnp.concatenate([e1, e2, e3], axis=-1).astype(bf16)


def _rep(shape):
    nd = len(shape)
    return pl.BlockSpec(shape, lambda i, _n=nd: (0,) * _n)


def _rows(shape):
    rest = shape[1:]
    return pl.BlockSpec((BLK,) + rest,
                        lambda i, _n=len(rest): (i,) + (0,) * _n)


@jax.jit
def kernel(x, enc_W1, enc_b1, enc_W2, enc_b2, mu_W, mu_b, lv_W, lv_b,
           codebooks, dec_W1, dec_b1, dec_W2, dec_b2, dec_W3, dec_b3, eps):
    outs = (
        jax.ShapeDtypeStruct((B, INPUT_DIM), f32),
        jax.ShapeDtypeStruct((B, LATENT), f32),
        jax.ShapeDtypeStruct((B, LATENT), f32),
        jax.ShapeDtypeStruct((B, LATENT), f32),
        jax.ShapeDtypeStruct((B, NQ), jnp.int32),
    )
    cbT = codebooks.transpose(0, 2, 1)
    cbTh = cbT.astype(bf16)
    cbTl = (cbT - cbTh.astype(f32)).astype(bf16)
    cbs = _split3(codebooks)
    en = jnp.sum(codebooks * codebooks, axis=2)[:, None, :]
    b1 = enc_b1.reshape(1, -1)
    b2 = enc_b2.reshape(1, -1)
    mub = mu_b.reshape(1, -1)
    lvb = lv_b.reshape(1, -1)
    db1 = dec_b1.reshape(1, -1)
    db2 = dec_b2.reshape(1, -1)
    db3 = dec_b3.reshape(1, -1)
    in_specs = [
        _rows(x.shape),
        _rep(enc_W1.shape), _rep(b1.shape), _rep(enc_W2.shape), _rep(b2.shape),
        _rep(mu_W.shape), _rep(mub.shape), _rep(lv_W.shape), _rep(lvb.shape),
        _rep(cbTh.shape), _rep(cbTl.shape), _rep(cbs.shape), _rep(en.shape),
        _rep(dec_W1.shape), _rep(db1.shape), _rep(dec_W2.shape), _rep(db2.shape),
        _rep(dec_W3.shape), _rep(db3.shape),
        _rows(eps.shape),
    ]
    out_specs = tuple(_rows(o.shape) for o in outs)
    return pl.pallas_call(
        _body,
        grid=(B // BLK,),
        in_specs=in_specs,
        out_specs=out_specs,
        out_shape=outs,
    )(x, enc_W1, b1, enc_W2, b2, mu_W, mub, lv_W, lvb,
      cbTh, cbTl, cbs, en, dec_W1, db1, dec_W2, db2, dec_W3, db3, eps)
